# baseline (speedup 1.0000x reference)
import jax
import jax.numpy as jnp
from jax import lax
from jax.experimental import pallas as pl
from jax.experimental.pallas import tpu as pltpu
from jax.experimental.pallas import tpu_sc as plsc

N_FACTORS = 64
BATCH = 16384
NC = 2
NS = 16
NW = NC * NS
B_PER_W = BATCH // NW
GROUPS = B_PER_W // 16


def _sc_body(data_hbm, uf_hbm, if_hbm, out_hbm,
             pairs_v, u_rows, v_rows, out_buf, sem_u, sem_v):
    wid = lax.axis_index("s") * NC + lax.axis_index("c")
    base = wid * B_PER_W

    pltpu.sync_copy(data_hbm.at[pl.ds(base, B_PER_W)], pairs_v)

    lane = lax.iota(jnp.int32, 16)
    zero = jnp.zeros((16,), jnp.int32)
    one = zero + 1

    copies = []
    for g in range(GROUPS):
        rows = lane + (g * 16)
        iu = plsc.load_gather(pairs_v, [rows, zero])
        iv = plsc.load_gather(pairs_v, [rows, one])
        dst = pl.ds(g * 16, 16)
        copies.append(pltpu.async_copy(uf_hbm.at[iu], u_rows.at[dst], sem_u))
        copies.append(pltpu.async_copy(if_hbm.at[iv], v_rows.at[dst], sem_v))
    for c in copies:
        c.wait()

    def gbody(g, _):
        rows = g * 16 + lane
        acc0 = jnp.zeros((16,), jnp.float32)
        acc1 = jnp.zeros((16,), jnp.float32)
        acc2 = jnp.zeros((16,), jnp.float32)
        acc3 = jnp.zeros((16,), jnp.float32)
        accs = [acc0, acc1, acc2, acc3]
        for d in range(N_FACTORS):
            col = jnp.full((16,), d, jnp.int32)
            u = plsc.load_gather(u_rows, [rows, col])
            v = plsc.load_gather(v_rows, [rows, col])
            accs[d % 4] = accs[d % 4] + u * v
        out_buf[g] = (accs[0] + accs[1]) + (accs[2] + accs[3])
        return 0

    lax.fori_loop(0, GROUPS, gbody, 0)

    pltpu.sync_copy(out_buf, out_hbm.at[pl.ds(wid * GROUPS, GROUPS)])


@jax.jit
def _mf_dot(data, user_factors, item_factors):
    mesh = plsc.VectorSubcoreMesh(
        core_axis_name="c", subcore_axis_name="s",
        num_cores=NC, num_subcores=NS)
    k = pl.kernel(
        _sc_body,
        out_type=jax.ShapeDtypeStruct((BATCH // 16, 16), jnp.float32),
        mesh=mesh,
        compiler_params=pltpu.CompilerParams(
            needs_layout_passes=False, use_tc_tiling_on_sc=False,
            disable_bounds_checks=True),
        scratch_types=[
            pltpu.VMEM((B_PER_W, 2), jnp.int32),
            pltpu.VMEM((B_PER_W, N_FACTORS), jnp.float32),
            pltpu.VMEM((B_PER_W, N_FACTORS), jnp.float32),
            pltpu.VMEM((GROUPS, 16), jnp.float32),
            pltpu.SemaphoreType.DMA,
            pltpu.SemaphoreType.DMA,
        ],
    )
    return k(data, user_factors, item_factors)


def kernel(data, user_factors, item_factors):
    out = _mf_dot(data.astype(jnp.int32), user_factors, item_factors)
    return out.reshape(BATCH)


# speedup vs baseline: 1.0028x; 1.0028x over previous
import jax
import jax.numpy as jnp
from jax import lax
from jax.experimental import pallas as pl
from jax.experimental.pallas import tpu as pltpu
from jax.experimental.pallas import tpu_sc as plsc

N_FACTORS = 64
BATCH = 16384
NC = 2
NS = 16
NW = NC * NS
B_PER_W = BATCH // NW
GROUPS = B_PER_W // 16


def _sc_body(data_hbm, uf_hbm, if_hbm, out_hbm,
             pairs_v, u_rows, v_rows, out_buf, sem_u, sem_v):
    wid = lax.axis_index("s") * NC + lax.axis_index("c")
    base = wid * B_PER_W

    with jax.named_scope("stage_idx"):
        pltpu.sync_copy(data_hbm.at[pl.ds(base, B_PER_W)], pairs_v)

    lane = lax.iota(jnp.int32, 16)
    zero = jnp.zeros((16,), jnp.int32)
    one = zero + 1

    with jax.named_scope("fire_gathers"):
        copies = []
        for g in range(GROUPS):
            rows = lane + (g * 16)
            iu = plsc.load_gather(pairs_v, [rows, zero])
            iv = plsc.load_gather(pairs_v, [rows, one])
            dst = pl.ds(g * 16, 16)
            copies.append(pltpu.async_copy(uf_hbm.at[iu], u_rows.at[dst], sem_u))
            copies.append(pltpu.async_copy(if_hbm.at[iv], v_rows.at[dst], sem_v))
    with jax.named_scope("wait_gathers"):
        for c in copies:
            c.wait()

    def gbody(g, _):
        rows = g * 16 + lane
        acc0 = jnp.zeros((16,), jnp.float32)
        acc1 = jnp.zeros((16,), jnp.float32)
        acc2 = jnp.zeros((16,), jnp.float32)
        acc3 = jnp.zeros((16,), jnp.float32)
        accs = [acc0, acc1, acc2, acc3]
        for d in range(N_FACTORS):
            col = jnp.full((16,), d, jnp.int32)
            u = plsc.load_gather(u_rows, [rows, col])
            v = plsc.load_gather(v_rows, [rows, col])
            accs[d % 4] = accs[d % 4] + u * v
        out_buf[g] = (accs[0] + accs[1]) + (accs[2] + accs[3])
        return 0

    with jax.named_scope("compute"):
        lax.fori_loop(0, GROUPS, gbody, 0)

    with jax.named_scope("out_copy"):
        pltpu.sync_copy(out_buf, out_hbm.at[pl.ds(wid * GROUPS, GROUPS)])


@jax.jit
def _mf_dot(data, user_factors, item_factors):
    mesh = plsc.VectorSubcoreMesh(
        core_axis_name="c", subcore_axis_name="s",
        num_cores=NC, num_subcores=NS)
    k = pl.kernel(
        _sc_body,
        out_type=jax.ShapeDtypeStruct((BATCH // 16, 16), jnp.float32),
        mesh=mesh,
        compiler_params=pltpu.CompilerParams(
            needs_layout_passes=False, use_tc_tiling_on_sc=False,
            disable_bounds_checks=True),
        scratch_types=[
            pltpu.VMEM((B_PER_W, 2), jnp.int32),
            pltpu.VMEM((B_PER_W, N_FACTORS), jnp.float32),
            pltpu.VMEM((B_PER_W, N_FACTORS), jnp.float32),
            pltpu.VMEM((GROUPS, 16), jnp.float32),
            pltpu.SemaphoreType.DMA,
            pltpu.SemaphoreType.DMA,
        ],
    )
    return k(data, user_factors, item_factors)


def kernel(data, user_factors, item_factors):
    out = _mf_dot(data.astype(jnp.int32), user_factors, item_factors)
    return out.reshape(BATCH)


# speedup vs baseline: 1.2409x; 1.2374x over previous
import jax
import jax.numpy as jnp
from jax import lax
from jax.experimental import pallas as pl
from jax.experimental.pallas import tpu as pltpu
from jax.experimental.pallas import tpu_sc as plsc

N_FACTORS = 64
BATCH = 16384
NC = 2
NS = 16
NW = NC * NS
B_PER_W = BATCH // NW
GROUPS = B_PER_W // 16
BLOCKS = BATCH // 128
BLK_PER_W = BLOCKS // NW


def _sc_body(data_hbm, uf_hbm, if_hbm, out_hbm,
             pairs_v, u_rows, v_rows, out_buf, sem_u, sem_v):
    wid = lax.axis_index("s") * NC + lax.axis_index("c")

    pltpu.sync_copy(data_hbm.at[pl.ds(wid * BLK_PER_W, BLK_PER_W)], pairs_v)

    def fire(g, _):
        t = lax.shift_right_logical(g, 3)
        k = jnp.bitwise_and(g, 7)
        iu = pairs_v[t, 0, pl.ds(k * 16, 16)]
        iv = pairs_v[t, 1, pl.ds(k * 16, 16)]
        dst = pl.ds(g * 16, 16)
        pltpu.async_copy(uf_hbm.at[iu], u_rows.at[dst], sem_u)
        pltpu.async_copy(if_hbm.at[iv], v_rows.at[dst], sem_v)
        return 0

    lax.fori_loop(0, GROUPS, fire, 0)

    pltpu.make_async_copy(uf_hbm.at[pl.ds(0, B_PER_W)], u_rows, sem_u).wait()
    pltpu.make_async_copy(if_hbm.at[pl.ds(0, B_PER_W)], v_rows, sem_v).wait()

    lane = lax.iota(jnp.int32, 16)
    last = lane == 15

    def row_body(b, _):
        u0 = u_rows[b, pl.ds(0, 16)]
        u1 = u_rows[b, pl.ds(16, 16)]
        u2 = u_rows[b, pl.ds(32, 16)]
        u3 = u_rows[b, pl.ds(48, 16)]
        v0 = v_rows[b, pl.ds(0, 16)]
        v1 = v_rows[b, pl.ds(16, 16)]
        v2 = v_rows[b, pl.ds(32, 16)]
        v3 = v_rows[b, pl.ds(48, 16)]
        p = (u0 * v0 + u1 * v1) + (u2 * v2 + u3 * v3)
        s = lax.cumsum(p)
        plsc.store_scatter(out_buf, [jnp.zeros((16,), jnp.int32) + b], s,
                           mask=last)
        return 0

    lax.fori_loop(0, B_PER_W, row_body, 0)

    pltpu.sync_copy(out_buf, out_hbm.at[pl.ds(wid * B_PER_W, B_PER_W)])


@jax.jit
def _mf_dot(data3, user_factors, item_factors):
    mesh = plsc.VectorSubcoreMesh(
        core_axis_name="c", subcore_axis_name="s",
        num_cores=NC, num_subcores=NS)
    k = pl.kernel(
        _sc_body,
        out_type=jax.ShapeDtypeStruct((BATCH,), jnp.float32),
        mesh=mesh,
        compiler_params=pltpu.CompilerParams(
            needs_layout_passes=False, use_tc_tiling_on_sc=False,
            disable_bounds_checks=True),
        scratch_types=[
            pltpu.VMEM((BLK_PER_W, 2, 128), jnp.int32),
            pltpu.VMEM((B_PER_W, N_FACTORS), jnp.float32),
            pltpu.VMEM((B_PER_W, N_FACTORS), jnp.float32),
            pltpu.VMEM((B_PER_W,), jnp.float32),
            pltpu.SemaphoreType.DMA,
            pltpu.SemaphoreType.DMA,
        ],
    )
    return k(data3, user_factors, item_factors)


def kernel(data, user_factors, item_factors):
    data3 = data.astype(jnp.int32).reshape(BLOCKS, 128, 2).transpose(0, 2, 1)
    return _mf_dot(data3, user_factors, item_factors)


# speedup vs baseline: 1.2433x; 1.0020x over previous
import jax
import jax.numpy as jnp
from jax import lax
from jax.experimental import pallas as pl
from jax.experimental.pallas import tpu as pltpu
from jax.experimental.pallas import tpu_sc as plsc

N_FACTORS = 64
BATCH = 16384
NC = 2
NS = 16
NW = NC * NS
B_PER_W = BATCH // NW
GROUPS = B_PER_W // 16
BLOCKS = BATCH // 128
BLK_PER_W = BLOCKS // NW


def _sc_body(data_hbm, uf_hbm, if_hbm, out_hbm,
             pairs_v, u_rows, v_rows, out_buf, sem_u, sem_v):
    wid = lax.axis_index("s") * NC + lax.axis_index("c")

    with jax.named_scope("stage_idx"):
        pltpu.sync_copy(data_hbm.at[pl.ds(wid * BLK_PER_W, BLK_PER_W)], pairs_v)

    def fire(g, _):
        t = lax.shift_right_logical(g, 3)
        k = jnp.bitwise_and(g, 7)
        iu = pairs_v[t, 0, pl.ds(k * 16, 16)]
        iv = pairs_v[t, 1, pl.ds(k * 16, 16)]
        dst = pl.ds(g * 16, 16)
        pltpu.async_copy(uf_hbm.at[iu], u_rows.at[dst], sem_u)
        pltpu.async_copy(if_hbm.at[iv], v_rows.at[dst], sem_v)
        return 0

    with jax.named_scope("fire_gathers"):
        lax.fori_loop(0, GROUPS, fire, 0)

    with jax.named_scope("wait_gathers"):
        pltpu.make_async_copy(uf_hbm.at[pl.ds(0, B_PER_W)], u_rows, sem_u).wait()
        pltpu.make_async_copy(if_hbm.at[pl.ds(0, B_PER_W)], v_rows, sem_v).wait()

    lane = lax.iota(jnp.int32, 16)
    last = lane == 15

    def row_body(b, _):
        u0 = u_rows[b, pl.ds(0, 16)]
        u1 = u_rows[b, pl.ds(16, 16)]
        u2 = u_rows[b, pl.ds(32, 16)]
        u3 = u_rows[b, pl.ds(48, 16)]
        v0 = v_rows[b, pl.ds(0, 16)]
        v1 = v_rows[b, pl.ds(16, 16)]
        v2 = v_rows[b, pl.ds(32, 16)]
        v3 = v_rows[b, pl.ds(48, 16)]
        p = (u0 * v0 + u1 * v1) + (u2 * v2 + u3 * v3)
        s = lax.cumsum(p)
        plsc.store_scatter(out_buf, [jnp.zeros((16,), jnp.int32) + b], s,
                           mask=last)
        return 0

    with jax.named_scope("compute"):
        lax.fori_loop(0, B_PER_W, row_body, 0)

    with jax.named_scope("out_copy"):
        pltpu.sync_copy(out_buf, out_hbm.at[pl.ds(wid * B_PER_W, B_PER_W)])


@jax.jit
def _mf_dot(data3, user_factors, item_factors):
    mesh = plsc.VectorSubcoreMesh(
        core_axis_name="c", subcore_axis_name="s",
        num_cores=NC, num_subcores=NS)
    k = pl.kernel(
        _sc_body,
        out_type=jax.ShapeDtypeStruct((BATCH,), jnp.float32),
        mesh=mesh,
        compiler_params=pltpu.CompilerParams(
            needs_layout_passes=False, use_tc_tiling_on_sc=False,
            disable_bounds_checks=True),
        scratch_types=[
            pltpu.VMEM((BLK_PER_W, 2, 128), jnp.int32),
            pltpu.VMEM((B_PER_W, N_FACTORS), jnp.float32),
            pltpu.VMEM((B_PER_W, N_FACTORS), jnp.float32),
            pltpu.VMEM((B_PER_W,), jnp.float32),
            pltpu.SemaphoreType.DMA,
            pltpu.SemaphoreType.DMA,
        ],
    )
    return k(data3, user_factors, item_factors)


def kernel(data, user_factors, item_factors):
    data3 = data.astype(jnp.int32).reshape(BLOCKS, 128, 2).transpose(0, 2, 1)
    return _mf_dot(data3, user_factors, item_factors)
